# baseline (speedup 1.0000x reference)
import functools

import jax
import jax.numpy as jnp
from jax import lax
from jax.experimental import pallas as pl
from jax.experimental.pallas import tpu as pltpu
from jax.experimental.pallas import tpu_sc as plsc

EMBED_DIM = 768
NUM_HEADS = 12
HEAD_DIM = 64
LENGTH = 5
KER = 17
POOL = 50
TOP_K = 5
B = 64
S = 197
HALF = LENGTH + KER - 1
WID = HEAD_DIM + KER - 1
PATCH = KER * KER
ROW = LENGTH * HEAD_DIM

_BB = 16
_NB = B // _BB
_UNITS = 2 * NUM_HEADS
_UPS = _UNITS // _NB


def _fused_body(x_ref, w1_ref, b1_ref, w2_ref, b2_ref, pk_ref,
                kv_ref, w_ref, b_ref,
                sim_ref, val_ref, idx_ref, rsum_ref, j_ref, gen_ref):
    xm = jnp.sum(x_ref[...], axis=1) * (1.0 / S)
    h = jnp.maximum(jnp.dot(xm, w1_ref[...],
                            preferred_element_type=jnp.float32) + b1_ref[...], 0.0)
    xq = jnp.dot(h, w2_ref[...], preferred_element_type=jnp.float32) + b2_ref[...]
    xn = xq * lax.rsqrt(jnp.maximum(jnp.sum(xq * xq, axis=-1, keepdims=True), 1e-12))
    pk = pk_ref[...]
    pkn = pk * lax.rsqrt(jnp.maximum(jnp.sum(pk * pk, axis=-1, keepdims=True), 1e-12))
    logits = lax.dot_general(xn, pkn, (((1,), (1,)), ((), ())),
                             preferred_element_type=jnp.float32)
    sim_ref[...] = jax.nn.sigmoid(logits)
    iota = lax.broadcasted_iota(jnp.int32, (_BB, POOL), 1)
    cur = logits
    vals = []
    idxs = []
    for _ in range(TOP_K):
        m = jnp.max(cur, axis=1, keepdims=True)
        am = jnp.min(jnp.where(cur == m, iota, POOL), axis=1, keepdims=True)
        vals.append(m)
        idxs.append(am)
        cur = jnp.where(iota == am, -jnp.inf, cur)
    v = jnp.concatenate(vals, axis=1)
    val_ref[...] = jax.nn.sigmoid(v)
    idxb = jnp.concatenate(idxs, axis=1)
    idx_ref[...] = idxb
    iota60 = lax.broadcasted_iota(jnp.int32, (_BB, NUM_HEADS * TOP_K), 1)
    j_ref[...] = jnp.concatenate([idxb] * NUM_HEADS, axis=1) + (iota60 // TOP_K) * POOL
    part = jnp.sum(v).reshape(1, 1) * (1.0 / B)
    @pl.when(pl.program_id(0) == 0)
    def _():
        rsum_ref[...] = jnp.zeros_like(rsum_ref)
    rsum_ref[...] += part

    for j in range(_UPS):
        kp = kv_ref[j]
        w = w_ref[j]
        bias = b_ref[j]
        for l in range(LENGTH):
            win = kp[l:l + KER, :]
            cols = [win[:, c:c + HEAD_DIM] for c in range(KER)]
            q = jnp.stack(cols, axis=1).reshape(PATCH, HEAD_DIM)
            res = jnp.dot(w, q, preferred_element_type=jnp.float32) + bias
            gen_ref[j // 2, :, j % 2, l, :] = res


_NPAIR = B * NUM_HEADS * TOP_K
_PER_TILE = _NPAIR // 32


_HA = 64
_HB = _PER_TILE - _HA


def _sc_gather_body(j_hbm, gen_hbm, out_hbm, idx_v, rows_v,
                    sem_a, sem_b, sem_w0, sem_w1):
    wid = lax.axis_index("s") * 2 + lax.axis_index("c")
    base = wid * _PER_TILE
    pltpu.sync_copy(j_hbm.at[pl.ds(base, _PER_TILE)], idx_v)
    ga = pltpu.async_copy(gen_hbm.at[idx_v.at[pl.ds(0, _HA)]],
                          rows_v.at[pl.ds(0, _HA)], sem_a)
    gb = pltpu.async_copy(gen_hbm.at[idx_v.at[pl.ds(_HA, _HB)]],
                          rows_v.at[pl.ds(_HA, _HB)], sem_b)
    ga.wait()
    wa0 = pltpu.async_copy(rows_v.at[pl.ds(0, _HA), pl.ds(0, ROW)],
                           out_hbm.at[pl.ds(base, _HA)], sem_w0)
    wa1 = pltpu.async_copy(rows_v.at[pl.ds(0, _HA), pl.ds(ROW, ROW)],
                           out_hbm.at[pl.ds(_NPAIR + base, _HA)], sem_w1)
    gb.wait()
    wa0.wait()
    wa1.wait()
    pltpu.sync_copy(rows_v.at[pl.ds(_HA, _HB), pl.ds(0, ROW)],
                    out_hbm.at[pl.ds(base + _HA, _HB)])
    pltpu.sync_copy(rows_v.at[pl.ds(_HA, _HB), pl.ds(ROW, ROW)],
                    out_hbm.at[pl.ds(_NPAIR + base + _HA, _HB)])


def kernel(x_embed, prompt, prompt_key, W1, b1, W2, b2, ck_w, ck_b, cv_w, cv_b,
           layer_num):
    layer_prompt = lax.dynamic_index_in_dim(prompt, layer_num, 0, keepdims=False)
    k_part = layer_prompt[:HALF].reshape(HALF, NUM_HEADS, WID).transpose(1, 0, 2)
    v_part = layer_prompt[HALF:].reshape(HALF, NUM_HEADS, WID).transpose(1, 0, 2)
    kv = jnp.stack([k_part, v_part], axis=1).reshape(_UNITS, HALF, WID)
    w_kv = jnp.stack([ck_w.reshape(POOL, NUM_HEADS, PATCH).transpose(1, 0, 2),
                      cv_w.reshape(POOL, NUM_HEADS, PATCH).transpose(1, 0, 2)],
                     axis=1).reshape(_UNITS, POOL, PATCH)
    b_kv = jnp.stack([ck_b.T, cv_b.T], axis=1).reshape(_UNITS, POOL)[..., None]

    sim, vals, idx, rsum, j2, gen = pl.pallas_call(
        _fused_body,
        grid=(_NB,),
        in_specs=[
            pl.BlockSpec((_BB, S, EMBED_DIM), lambda i: (i, 0, 0)),
            pl.BlockSpec((EMBED_DIM, EMBED_DIM // 2), lambda i: (0, 0)),
            pl.BlockSpec((1, EMBED_DIM // 2), lambda i: (0, 0)),
            pl.BlockSpec((EMBED_DIM // 2, EMBED_DIM // 4), lambda i: (0, 0)),
            pl.BlockSpec((1, EMBED_DIM // 4), lambda i: (0, 0)),
            pl.BlockSpec((POOL, EMBED_DIM // 4), lambda i: (0, 0)),
            pl.BlockSpec((_UPS, HALF, WID), lambda i: (i, 0, 0)),
            pl.BlockSpec((_UPS, POOL, PATCH), lambda i: (i, 0, 0)),
            pl.BlockSpec((_UPS, POOL, 1), lambda i: (i, 0, 0)),
        ],
        out_specs=[
            pl.BlockSpec((_BB, POOL), lambda i: (i, 0)),
            pl.BlockSpec((_BB, TOP_K), lambda i: (i, 0)),
            pl.BlockSpec((_BB, TOP_K), lambda i: (i, 0)),
            pl.BlockSpec((1, 1), lambda i: (0, 0)),
            pl.BlockSpec((_BB, NUM_HEADS * TOP_K), lambda i: (i, 0)),
            pl.BlockSpec((_UPS // 2, POOL, 2, LENGTH, HEAD_DIM),
                         lambda i: (i, 0, 0, 0, 0)),
        ],
        out_shape=[
            jax.ShapeDtypeStruct((B, POOL), jnp.float32),
            jax.ShapeDtypeStruct((B, TOP_K), jnp.float32),
            jax.ShapeDtypeStruct((B, TOP_K), jnp.int32),
            jax.ShapeDtypeStruct((1, 1), jnp.float32),
            jax.ShapeDtypeStruct((B, NUM_HEADS * TOP_K), jnp.int32),
            jax.ShapeDtypeStruct((NUM_HEADS, POOL, 2, LENGTH, HEAD_DIM),
                                 jnp.float32),
        ],
    )(x_embed, W1, b1.reshape(1, -1), W2, b2.reshape(1, -1), prompt_key,
      kv, w_kv, b_kv)

    gather = functools.partial(
        pl.kernel,
        out_type=jax.ShapeDtypeStruct((2 * _NPAIR, ROW), jnp.float32),
        mesh=plsc.VectorSubcoreMesh(core_axis_name="c", subcore_axis_name="s"),
        compiler_params=pltpu.CompilerParams(use_tc_tiling_on_sc=False),
        scratch_types=[
            pltpu.VMEM((_PER_TILE,), jnp.int32),
            pltpu.VMEM((_PER_TILE, 2 * ROW), jnp.float32),
            pltpu.SemaphoreType.DMA,
            pltpu.SemaphoreType.DMA,
            pltpu.SemaphoreType.DMA,
            pltpu.SemaphoreType.DMA,
        ],
    )(_sc_gather_body)
    out_all = gather(j2.reshape(_NPAIR), gen.reshape(NUM_HEADS * POOL, 2 * ROW))

    batched_prompt = out_all.reshape(2, B, NUM_HEADS, TOP_K * LENGTH, HEAD_DIM)
    return batched_prompt, rsum.reshape(()), sim, vals, idx


# speedup vs baseline: 1.0156x; 1.0156x over previous
import functools

import jax
import jax.numpy as jnp
from jax import lax
from jax.experimental import pallas as pl
from jax.experimental.pallas import tpu as pltpu
from jax.experimental.pallas import tpu_sc as plsc

EMBED_DIM = 768
NUM_HEADS = 12
HEAD_DIM = 64
LENGTH = 5
KER = 17
POOL = 50
TOP_K = 5
B = 64
S = 197
HALF = LENGTH + KER - 1
WID = HEAD_DIM + KER - 1
PATCH = KER * KER
ROW = LENGTH * HEAD_DIM

_BB = 16
_NB = B // _BB
_UNITS = 2 * NUM_HEADS
_UPS = _UNITS // _NB


def _fused_body(x_ref, w1_ref, b1_ref, w2_ref, b2_ref, pk_ref,
                kv_ref, w_ref, b_ref,
                sim_ref, val_ref, idx_ref, rsum_ref, j_ref, gen_ref):
    xm = jnp.sum(x_ref[...], axis=1) * (1.0 / S)
    h = jnp.maximum(jnp.dot(xm, w1_ref[...],
                            preferred_element_type=jnp.float32) + b1_ref[...], 0.0)
    xq = jnp.dot(h, w2_ref[...], preferred_element_type=jnp.float32) + b2_ref[...]
    xn = xq * lax.rsqrt(jnp.maximum(jnp.sum(xq * xq, axis=-1, keepdims=True), 1e-12))
    pk = pk_ref[...]
    pkn = pk * lax.rsqrt(jnp.maximum(jnp.sum(pk * pk, axis=-1, keepdims=True), 1e-12))
    logits = lax.dot_general(xn, pkn, (((1,), (1,)), ((), ())),
                             preferred_element_type=jnp.float32)
    sim_ref[...] = jax.nn.sigmoid(logits)
    iota = lax.broadcasted_iota(jnp.int32, (_BB, POOL), 1)
    cur = logits
    vals = []
    idxs = []
    for _ in range(TOP_K):
        m = jnp.max(cur, axis=1, keepdims=True)
        am = jnp.min(jnp.where(cur == m, iota, POOL), axis=1, keepdims=True)
        vals.append(m)
        idxs.append(am)
        cur = jnp.where(iota == am, -jnp.inf, cur)
    v = jnp.concatenate(vals, axis=1)
    val_ref[...] = jax.nn.sigmoid(v)
    idxb = jnp.concatenate(idxs, axis=1)
    idx_ref[...] = idxb
    iota60 = lax.broadcasted_iota(jnp.int32, (_BB, NUM_HEADS * TOP_K), 1)
    j_ref[...] = jnp.concatenate([idxb] * NUM_HEADS, axis=1) + (iota60 // TOP_K) * POOL
    part = jnp.sum(v).reshape(1, 1) * (1.0 / B)
    @pl.when(pl.program_id(0) == 0)
    def _():
        rsum_ref[...] = jnp.zeros_like(rsum_ref)
    rsum_ref[...] += part

    for j in range(_UPS):
        kp = kv_ref[j]
        w = w_ref[j]
        bias = b_ref[j]
        for l in range(LENGTH):
            win = kp[l:l + KER, :]
            cols = [win[:, c:c + HEAD_DIM] for c in range(KER)]
            q = jnp.stack(cols, axis=1).reshape(PATCH, HEAD_DIM)
            res = jnp.dot(w, q, preferred_element_type=jnp.float32) + bias
            gen_ref[j // 2, :, j % 2, l, :] = res


_NPAIR = B * NUM_HEADS * TOP_K
_PER_TILE = _NPAIR // 32


def _sc_gather_body(j_hbm, gen_hbm, out_hbm, idx_v, rows_v,
                    sem, sem_w0, sem_w1):
    wid = lax.axis_index("s") * 2 + lax.axis_index("c")
    base = wid * _PER_TILE
    pltpu.sync_copy(j_hbm.at[pl.ds(base, _PER_TILE)], idx_v)
    pltpu.async_copy(gen_hbm.at[idx_v], rows_v, sem).wait()
    w0 = pltpu.async_copy(rows_v.at[:, pl.ds(0, ROW)],
                          out_hbm.at[pl.ds(base, _PER_TILE)], sem_w0)
    w1 = pltpu.async_copy(rows_v.at[:, pl.ds(ROW, ROW)],
                          out_hbm.at[pl.ds(_NPAIR + base, _PER_TILE)], sem_w1)
    w0.wait()
    w1.wait()


def kernel(x_embed, prompt, prompt_key, W1, b1, W2, b2, ck_w, ck_b, cv_w, cv_b,
           layer_num):
    layer_prompt = lax.dynamic_index_in_dim(prompt, layer_num, 0, keepdims=False)
    k_part = layer_prompt[:HALF].reshape(HALF, NUM_HEADS, WID).transpose(1, 0, 2)
    v_part = layer_prompt[HALF:].reshape(HALF, NUM_HEADS, WID).transpose(1, 0, 2)
    kv = jnp.stack([k_part, v_part], axis=1).reshape(_UNITS, HALF, WID)
    w_kv = jnp.stack([ck_w.reshape(POOL, NUM_HEADS, PATCH).transpose(1, 0, 2),
                      cv_w.reshape(POOL, NUM_HEADS, PATCH).transpose(1, 0, 2)],
                     axis=1).reshape(_UNITS, POOL, PATCH)
    b_kv = jnp.stack([ck_b.T, cv_b.T], axis=1).reshape(_UNITS, POOL)[..., None]

    sim, vals, idx, rsum, j2, gen = pl.pallas_call(
        _fused_body,
        grid=(_NB,),
        in_specs=[
            pl.BlockSpec((_BB, S, EMBED_DIM), lambda i: (i, 0, 0)),
            pl.BlockSpec((EMBED_DIM, EMBED_DIM // 2), lambda i: (0, 0)),
            pl.BlockSpec((1, EMBED_DIM // 2), lambda i: (0, 0)),
            pl.BlockSpec((EMBED_DIM // 2, EMBED_DIM // 4), lambda i: (0, 0)),
            pl.BlockSpec((1, EMBED_DIM // 4), lambda i: (0, 0)),
            pl.BlockSpec((POOL, EMBED_DIM // 4), lambda i: (0, 0)),
            pl.BlockSpec((_UPS, HALF, WID), lambda i: (i, 0, 0)),
            pl.BlockSpec((_UPS, POOL, PATCH), lambda i: (i, 0, 0)),
            pl.BlockSpec((_UPS, POOL, 1), lambda i: (i, 0, 0)),
        ],
        out_specs=[
            pl.BlockSpec((_BB, POOL), lambda i: (i, 0)),
            pl.BlockSpec((_BB, TOP_K), lambda i: (i, 0)),
            pl.BlockSpec((_BB, TOP_K), lambda i: (i, 0)),
            pl.BlockSpec((1, 1), lambda i: (0, 0)),
            pl.BlockSpec((_BB, NUM_HEADS * TOP_K), lambda i: (i, 0)),
            pl.BlockSpec((_UPS // 2, POOL, 2, LENGTH, HEAD_DIM),
                         lambda i: (i, 0, 0, 0, 0)),
        ],
        out_shape=[
            jax.ShapeDtypeStruct((B, POOL), jnp.float32),
            jax.ShapeDtypeStruct((B, TOP_K), jnp.float32),
            jax.ShapeDtypeStruct((B, TOP_K), jnp.int32),
            jax.ShapeDtypeStruct((1, 1), jnp.float32),
            jax.ShapeDtypeStruct((B, NUM_HEADS * TOP_K), jnp.int32),
            jax.ShapeDtypeStruct((NUM_HEADS, POOL, 2, LENGTH, HEAD_DIM),
                                 jnp.float32),
        ],
    )(x_embed, W1, b1.reshape(1, -1), W2, b2.reshape(1, -1), prompt_key,
      kv, w_kv, b_kv)

    gather = functools.partial(
        pl.kernel,
        out_type=jax.ShapeDtypeStruct((2 * _NPAIR, ROW), jnp.float32),
        mesh=plsc.VectorSubcoreMesh(core_axis_name="c", subcore_axis_name="s"),
        compiler_params=pltpu.CompilerParams(use_tc_tiling_on_sc=False),
        scratch_types=[
            pltpu.VMEM((_PER_TILE,), jnp.int32),
            pltpu.VMEM((_PER_TILE, 2 * ROW), jnp.float32),
            pltpu.SemaphoreType.DMA,
            pltpu.SemaphoreType.DMA,
            pltpu.SemaphoreType.DMA,
        ],
    )(_sc_gather_body)
    out_all = gather(j2.reshape(_NPAIR), gen.reshape(NUM_HEADS * POOL, 2 * ROW))

    batched_prompt = out_all.reshape(2, B, NUM_HEADS, TOP_K * LENGTH, HEAD_DIM)
    return batched_prompt, rsum.reshape(()), sim, vals, idx
